# Initial kernel scaffold; baseline (speedup 1.0000x reference)
import jax
import jax.numpy as jnp
from jax.experimental import pallas as pl


def kernel(x, edge_index, W1, b1, beta2, beta3, beta4, W2, b2):
    raise NotImplementedError("write your pallas kernel here")


# speedup vs baseline: 15.2007x; 15.2007x over previous
import functools

import jax
import jax.numpy as jnp
from jax import lax
from jax.experimental import pallas as pl
from jax.experimental.pallas import tpu as pltpu
from jax.experimental.pallas import tpu_sc as plsc

_N = 10000
_E = 320000
_D = 128
_H = 16
_C = 16

_L = 16
_NT = 16
_RPT = 640
_N1 = _NT * _RPT
_B = 128
_NBATCH = 162
_EPT = _NBATCH * _B
_E1 = _NT * _EPT


def _rsqrt_newton(v):
    v = jnp.maximum(v, jnp.float32(1e-24))
    i = plsc.bitcast(v, jnp.int32)
    i = jnp.int32(0x5F3759DF) - lax.shift_right_logical(i, 1)
    y = plsc.bitcast(i, jnp.float32)
    for _ in range(3):
        y = y * (jnp.float32(1.5) - jnp.float32(0.5) * v * y * y)
    return y


def _conv_body(h_hbm, srcs_hbm, dsts_hbm, beta_hbm, out_hbm,
               src_t, dst_t, hbuf, nbuf, rows_s, rows_d, out_rows,
               ab, rsv, rdv, rbuf, dbuf, beta_v,
               rinv_spm, denom_spm, num_spm):
    w = lax.axis_index("s")
    base = w * _RPT
    lane = lax.iota(jnp.int32, _L)
    zrow = jnp.zeros((_L,), jnp.float32)

    pltpu.sync_copy(srcs_hbm.at[w], src_t)
    pltpu.sync_copy(dsts_hbm.at[w], dst_t)
    pltpu.sync_copy(beta_hbm, beta_v)

    pltpu.sync_copy(h_hbm.at[pl.ds(base, _RPT)], hbuf)

    def p0(rg, carry):
        acc = zrow
        for u in range(_L):
            r = rg * _L + u
            hr = hbuf[r, :]
            acc = jnp.where(lane == u, jnp.sum(hr * hr), acc)
            nbuf[r, :] = zrow
        rbuf[pl.ds(rg * _L, _L)] = _rsqrt_newton(acc)
        dbuf[pl.ds(rg * _L, _L)] = zrow
        return carry

    lax.fori_loop(0, _RPT // _L, p0, None)
    pltpu.sync_copy(rbuf, rinv_spm.at[pl.ds(base, _RPT)])
    pltpu.sync_copy(dbuf, denom_spm.at[pl.ds(base, _RPT)])
    pltpu.sync_copy(nbuf, num_spm.at[pl.ds(base, _RPT)])
    plsc.subcore_barrier()

    def p1(i, carry):
        si = src_t.at[i]
        di = dst_t.at[i]
        pltpu.sync_copy(h_hbm.at[si], rows_s)
        pltpu.sync_copy(h_hbm.at[di], rows_d)
        pltpu.sync_copy(rinv_spm.at[si], rsv)
        pltpu.sync_copy(rinv_spm.at[di], rdv)
        bv = beta_v[...]

        def grp(g, c):
            acc = zrow
            for u in range(_L):
                e = g * _L + u
                acc = jnp.where(lane == u,
                                jnp.sum(rows_s[e, :] * rows_d[e, :]), acc)
            off = g * _L
            alpha = acc * rsv[pl.ds(off, _L)] * rdv[pl.ds(off, _L)] * bv
            av = jnp.exp(alpha)
            ab[pl.ds(off, _L)] = av
            for u in range(_L):
                e = g * _L + u
                out_rows[e, :] = rows_s[e, :] * av[u]
            return c

        lax.fori_loop(0, _B // _L, grp, None)
        pltpu.sync_copy(ab, denom_spm.at[di], add=True)
        pltpu.sync_copy(out_rows, num_spm.at[di], add=True)
        return carry

    lax.fori_loop(0, _NBATCH, p1, None)
    plsc.subcore_barrier()

    pltpu.sync_copy(num_spm.at[pl.ds(base, _RPT)], nbuf)
    pltpu.sync_copy(denom_spm.at[pl.ds(base, _RPT)], dbuf)

    def p2(rg, carry):
        dv = dbuf[pl.ds(rg * _L, _L)]
        dinv = jnp.float32(1.0) / jnp.maximum(dv, jnp.float32(1e-30))
        for u in range(_L):
            r = rg * _L + u
            hbuf[r, :] = nbuf[r, :] * dinv[u]
        return carry

    lax.fori_loop(0, _RPT // _L, p2, None)
    pltpu.sync_copy(hbuf, out_hbm.at[pl.ds(base, _RPT)])


_conv = pl.kernel(
    _conv_body,
    out_type=jax.ShapeDtypeStruct((_N1, _H), jnp.float32),
    mesh=plsc.VectorSubcoreMesh(
        core_axis_name="c", subcore_axis_name="s", num_cores=1
    ),
    compiler_params=pltpu.CompilerParams(
        needs_layout_passes=False, use_tc_tiling_on_sc=False
    ),
    scratch_types=[
        pltpu.VMEM((_NBATCH, _B), jnp.int32),
        pltpu.VMEM((_NBATCH, _B), jnp.int32),
        pltpu.VMEM((_RPT, _H), jnp.float32),
        pltpu.VMEM((_RPT, _H), jnp.float32),
        pltpu.VMEM((_B, _H), jnp.float32),
        pltpu.VMEM((_B, _H), jnp.float32),
        pltpu.VMEM((_B, _H), jnp.float32),
        pltpu.VMEM((_B,), jnp.float32),
        pltpu.VMEM((_B,), jnp.float32),
        pltpu.VMEM((_B,), jnp.float32),
        pltpu.VMEM((_RPT,), jnp.float32),
        pltpu.VMEM((_RPT,), jnp.float32),
        pltpu.VMEM((_L,), jnp.float32),
        pltpu.VMEM_SHARED((_N1,), jnp.float32),
        pltpu.VMEM_SHARED((_N1,), jnp.float32),
        pltpu.VMEM_SHARED((_N1, _H), jnp.float32),
    ],
)


def _pre_body(x_ref, w_ref, b_ref, o_ref):
    acc = jnp.dot(x_ref[...], w_ref[...], preferred_element_type=jnp.float32)
    o_ref[...] = jnp.maximum(acc + b_ref[...], jnp.float32(0.0))


_pre = pl.pallas_call(
    _pre_body,
    grid=(10,),
    in_specs=[
        pl.BlockSpec((_N // 10, _D), lambda i: (i, 0)),
        pl.BlockSpec((_D, _H), lambda i: (0, 0)),
        pl.BlockSpec((1, _H), lambda i: (0, 0)),
    ],
    out_specs=pl.BlockSpec((_N // 10, _H), lambda i: (i, 0)),
    out_shape=jax.ShapeDtypeStruct((_N, _H), jnp.float32),
)


def _post_body(h_ref, w_ref, b_ref, o_ref):
    z = jnp.dot(h_ref[...], w_ref[...], preferred_element_type=jnp.float32)
    z = z + b_ref[...]
    z = z - jnp.max(z, axis=1, keepdims=True)
    o_ref[...] = z - jnp.log(jnp.sum(jnp.exp(z), axis=1, keepdims=True))


_post = pl.pallas_call(
    _post_body,
    grid=(10,),
    in_specs=[
        pl.BlockSpec((_N1 // 10, _H), lambda i: (i, 0)),
        pl.BlockSpec((_H, _C), lambda i: (0, 0)),
        pl.BlockSpec((1, _C), lambda i: (0, 0)),
    ],
    out_specs=pl.BlockSpec((_N1 // 10, _C), lambda i: (i, 0)),
    out_shape=jax.ShapeDtypeStruct((_N1, _C), jnp.float32),
)


def kernel(x, edge_index, W1, b1, beta2, beta3, beta4, W2, b2):
    h0 = _pre(x, W1, b1.reshape(1, _H))
    hp = jnp.concatenate(
        [h0, jnp.zeros((_N1 - _N, _H), jnp.float32)], axis=0
    )

    src = edge_index[0].astype(jnp.int32)
    dst = edge_index[1].astype(jnp.int32)
    loop = jnp.arange(_N, dtype=jnp.int32)
    pad = jnp.full((_E1 - _E - _N,), _N, dtype=jnp.int32)
    srcs = jnp.concatenate([src, loop, pad]).reshape(_NT, _NBATCH, _B)
    dsts = jnp.concatenate([dst, loop, pad]).reshape(_NT, _NBATCH, _B)

    ones = jnp.ones((_L,), jnp.float32)
    h = _conv(hp, srcs, dsts, ones)
    h = _conv(h, srcs, dsts, ones * beta2)
    h = _conv(h, srcs, dsts, ones * beta3)
    h = _conv(h, srcs, dsts, ones * beta4)

    out = _post(h, W2, b2.reshape(1, _C))
    return out[:_N]


# speedup vs baseline: 25.5745x; 1.6825x over previous
import functools

import jax
import jax.numpy as jnp
from jax import lax
from jax.experimental import pallas as pl
from jax.experimental.pallas import tpu as pltpu
from jax.experimental.pallas import tpu_sc as plsc

_N = 10000
_E = 320000
_D = 128
_H = 16
_C = 16

_L = 16
_NT = 16
_RPT = 640
_N1 = _NT * _RPT
_B = 128
_NBATCH = 162
_EPT = _NBATCH * _B
_E1 = _NT * _EPT


def _rsqrt_newton(v):
    v = jnp.maximum(v, jnp.float32(1e-24))
    i = plsc.bitcast(v, jnp.int32)
    i = jnp.int32(0x5F3759DF) - lax.shift_right_logical(i, 1)
    y = plsc.bitcast(i, jnp.float32)
    for _ in range(3):
        y = y * (jnp.float32(1.5) - jnp.float32(0.5) * v * y * y)
    return y


def _conv_body(h_hbm, srcs_hbm, dsts_hbm, beta_hbm, out_hbm,
               src_t, dst_t, hbuf, nbuf,
               rows_s0, rows_d0, rsv0, rdv0,
               rows_s1, rows_d1, rsv1, rdv1,
               out_rows, ab, rbuf, dbuf, beta_v,
               rinv_spm, denom_spm, num_spm):
    w = lax.axis_index("s")
    base = w * _RPT
    lane = lax.iota(jnp.int32, _L)
    zrow = jnp.zeros((_L,), jnp.float32)

    pltpu.sync_copy(srcs_hbm.at[w], src_t)
    pltpu.sync_copy(dsts_hbm.at[w], dst_t)
    pltpu.sync_copy(beta_hbm, beta_v)

    pltpu.sync_copy(h_hbm.at[pl.ds(base, _RPT)], hbuf)

    def p0(rg, carry):
        acc = zrow
        for u in range(_L):
            r = rg * _L + u
            hr = hbuf[r, :]
            acc = jnp.where(lane == u, jnp.sum(hr * hr), acc)
            nbuf[r, :] = zrow
        rbuf[pl.ds(rg * _L, _L)] = _rsqrt_newton(acc)
        dbuf[pl.ds(rg * _L, _L)] = zrow
        return carry

    lax.fori_loop(0, _RPT // _L, p0, None)
    pltpu.sync_copy(rbuf, rinv_spm.at[pl.ds(base, _RPT)])
    pltpu.sync_copy(dbuf, denom_spm.at[pl.ds(base, _RPT)])
    pltpu.sync_copy(nbuf, num_spm.at[pl.ds(base, _RPT)])
    plsc.subcore_barrier()

    bufs = ((rows_s0, rows_d0, rsv0, rdv0),
            (rows_s1, rows_d1, rsv1, rdv1))
    bv = beta_v[...]

    def issue_gathers(i, p, sem):
        rs, rd, vs, vd = bufs[p]
        si = src_t.at[i]
        di = dst_t.at[i]
        d = (pltpu.async_copy(h_hbm.at[si], rs, sem),
             pltpu.async_copy(h_hbm.at[di], rd, sem))
        pltpu.sync_copy(rinv_spm.at[si], vs)
        pltpu.sync_copy(rinv_spm.at[di], vd)
        return d

    def drain_gathers(descs):
        for d in descs:
            d.wait()

    def process(i, p):
        rs, rd, vs, vd = bufs[p]

        def grp(g, c):
            acc = zrow
            for u in range(_L):
                e = g * _L + u
                acc = jnp.where(lane == u,
                                jnp.sum(rs[e, :] * rd[e, :]), acc)
            off = g * _L
            alpha = acc * vs[pl.ds(off, _L)] * vd[pl.ds(off, _L)] * bv
            av = jnp.exp(alpha)
            ab[pl.ds(off, _L)] = av
            for u in range(_L):
                e = g * _L + u
                out_rows[e, :] = rs[e, :] * av[u]
            return c

        lax.fori_loop(0, _B // _L, grp, None)
        di = dst_t.at[i]
        pltpu.sync_copy(ab, denom_spm.at[di], add=True)
        pltpu.sync_copy(out_rows, num_spm.at[di], add=True)

    def p1_scoped(sem0, sem1):
        def p1(j, carry):
            b0 = j * 2
            d0 = issue_gathers(b0, 0, sem0)
            d1 = issue_gathers(b0 + 1, 1, sem1)
            drain_gathers(d0)
            process(b0, 0)
            drain_gathers(d1)
            process(b0 + 1, 1)
            return carry

        lax.fori_loop(0, _NBATCH // 2, p1, None)

    pl.run_scoped(p1_scoped,
                  sem0=pltpu.SemaphoreType.DMA(()),
                  sem1=pltpu.SemaphoreType.DMA(()))
    plsc.subcore_barrier()

    pltpu.sync_copy(num_spm.at[pl.ds(base, _RPT)], nbuf)
    pltpu.sync_copy(denom_spm.at[pl.ds(base, _RPT)], dbuf)

    def p2(rg, carry):
        dv = dbuf[pl.ds(rg * _L, _L)]
        dinv = jnp.float32(1.0) / jnp.maximum(dv, jnp.float32(1e-30))
        for u in range(_L):
            r = rg * _L + u
            hbuf[r, :] = nbuf[r, :] * dinv[u]
        return carry

    lax.fori_loop(0, _RPT // _L, p2, None)
    pltpu.sync_copy(hbuf, out_hbm.at[pl.ds(base, _RPT)])


_conv = pl.kernel(
    _conv_body,
    out_type=jax.ShapeDtypeStruct((_N1, _H), jnp.float32),
    mesh=plsc.VectorSubcoreMesh(
        core_axis_name="c", subcore_axis_name="s", num_cores=1
    ),
    compiler_params=pltpu.CompilerParams(
        needs_layout_passes=False, use_tc_tiling_on_sc=False
    ),
    scratch_types=[
        pltpu.VMEM((_NBATCH, _B), jnp.int32),
        pltpu.VMEM((_NBATCH, _B), jnp.int32),
        pltpu.VMEM((_RPT, _H), jnp.float32),
        pltpu.VMEM((_RPT, _H), jnp.float32),
        pltpu.VMEM((_B, _H), jnp.float32),
        pltpu.VMEM((_B, _H), jnp.float32),
        pltpu.VMEM((_B,), jnp.float32),
        pltpu.VMEM((_B,), jnp.float32),
        pltpu.VMEM((_B, _H), jnp.float32),
        pltpu.VMEM((_B, _H), jnp.float32),
        pltpu.VMEM((_B,), jnp.float32),
        pltpu.VMEM((_B,), jnp.float32),
        pltpu.VMEM((_B, _H), jnp.float32),
        pltpu.VMEM((_B,), jnp.float32),
        pltpu.VMEM((_RPT,), jnp.float32),
        pltpu.VMEM((_RPT,), jnp.float32),
        pltpu.VMEM((_L,), jnp.float32),
        pltpu.VMEM_SHARED((_N1,), jnp.float32),
        pltpu.VMEM_SHARED((_N1,), jnp.float32),
        pltpu.VMEM_SHARED((_N1, _H), jnp.float32),
    ],
)


def _pre_body(x_ref, w_ref, b_ref, o_ref):
    acc = jnp.dot(x_ref[...], w_ref[...], preferred_element_type=jnp.float32)
    o_ref[...] = jnp.maximum(acc + b_ref[...], jnp.float32(0.0))


_pre = pl.pallas_call(
    _pre_body,
    grid=(10,),
    in_specs=[
        pl.BlockSpec((_N // 10, _D), lambda i: (i, 0)),
        pl.BlockSpec((_D, _H), lambda i: (0, 0)),
        pl.BlockSpec((1, _H), lambda i: (0, 0)),
    ],
    out_specs=pl.BlockSpec((_N // 10, _H), lambda i: (i, 0)),
    out_shape=jax.ShapeDtypeStruct((_N, _H), jnp.float32),
)


def _post_body(h_ref, w_ref, b_ref, o_ref):
    z = jnp.dot(h_ref[...], w_ref[...], preferred_element_type=jnp.float32)
    z = z + b_ref[...]
    z = z - jnp.max(z, axis=1, keepdims=True)
    o_ref[...] = z - jnp.log(jnp.sum(jnp.exp(z), axis=1, keepdims=True))


_post = pl.pallas_call(
    _post_body,
    grid=(10,),
    in_specs=[
        pl.BlockSpec((_N1 // 10, _H), lambda i: (i, 0)),
        pl.BlockSpec((_H, _C), lambda i: (0, 0)),
        pl.BlockSpec((1, _C), lambda i: (0, 0)),
    ],
    out_specs=pl.BlockSpec((_N1 // 10, _C), lambda i: (i, 0)),
    out_shape=jax.ShapeDtypeStruct((_N1, _C), jnp.float32),
)


def kernel(x, edge_index, W1, b1, beta2, beta3, beta4, W2, b2):
    h0 = _pre(x, W1, b1.reshape(1, _H))
    hp = jnp.concatenate(
        [h0, jnp.zeros((_N1 - _N, _H), jnp.float32)], axis=0
    )

    src = edge_index[0].astype(jnp.int32)
    dst = edge_index[1].astype(jnp.int32)
    loop = jnp.arange(_N, dtype=jnp.int32)
    pad = jnp.full((_E1 - _E - _N,), _N, dtype=jnp.int32)
    srcs = jnp.concatenate([src, loop, pad]).reshape(_NT, _NBATCH, _B)
    dsts = jnp.concatenate([dst, loop, pad]).reshape(_NT, _NBATCH, _B)

    ones = jnp.ones((_L,), jnp.float32)
    h = _conv(hp, srcs, dsts, ones)
    h = _conv(h, srcs, dsts, ones * beta2)
    h = _conv(h, srcs, dsts, ones * beta3)
    h = _conv(h, srcs, dsts, ones * beta4)

    out = _post(h, W2, b2.reshape(1, _C))
    return out[:_N]
